# baseline (speedup 1.0000x reference)
import functools

import jax
import jax.numpy as jnp
from jax.experimental import pallas as pl
from jax.experimental.pallas import tpu as pltpu

_N = 65536
_D = 64
_K = 1024
_E = 80
_BN = 2048
_NB = _N // _BN
_NITERS = 10


def _iter_body(x_ref, acc_in_ref, clusters_ref, acc_out_ref, cent_ref, csq_ref):
    i = pl.program_id(0)

    @pl.when(i == 0)
    def _init():
        acc = acc_in_ref[...]
        cnt = acc[:, 64:65]
        cent = acc[:, 0:64] / cnt
        cent_ref[...] = cent
        csq_ref[...] = jnp.sum(cent * cent, axis=1)[None, :]

    xb = x_ref[...]
    xs = xb[:, 0:64]
    cent = cent_ref[...]
    x_sq = jnp.sum(xs * xs, axis=1, keepdims=True)
    xc = jax.lax.dot_general(xs, cent, (((1,), (1,)), ((), ())),
                             preferred_element_type=jnp.float32)
    scores = (x_sq - 2.0 * xc) + csq_ref[...]
    idx = jnp.argmin(scores, axis=1).astype(jnp.int32)
    clusters_ref[...] = idx
    oh = (jax.lax.broadcasted_iota(jnp.int32, (_BN, _K), 1) == idx[:, None])
    ohf = oh.astype(jnp.float32)
    pacc = jax.lax.dot_general(ohf, xb, (((0,), (0,)), ((), ())),
                               preferred_element_type=jnp.float32,
                               precision=jax.lax.Precision.HIGHEST)

    @pl.when(i == 0)
    def _first():
        acc_out_ref[...] = pacc

    @pl.when(i > 0)
    def _rest():
        acc_out_ref[...] += pacc


_iter_call = pl.pallas_call(
    _iter_body,
    grid=(_NB,),
    in_specs=[
        pl.BlockSpec((_BN, _E), lambda i: (i, 0)),
        pl.BlockSpec((_K, _E), lambda i: (0, 0)),
    ],
    out_specs=[
        pl.BlockSpec((_BN,), lambda i: (i,)),
        pl.BlockSpec((_K, _E), lambda i: (0, 0)),
    ],
    out_shape=[
        jax.ShapeDtypeStruct((_N,), jnp.int32),
        jax.ShapeDtypeStruct((_K, _E), jnp.float32),
    ],
    scratch_shapes=[
        pltpu.VMEM((_K, 64), jnp.float32),
        pltpu.VMEM((1, _K), jnp.float32),
    ],
    compiler_params=pltpu.CompilerParams(
        dimension_semantics=("arbitrary",),
    ),
)


def _final_body(acc_ref, cent_ref, npts_ref):
    acc = acc_ref[...]
    cnt = acc[:, 64:65]
    cent_ref[...] = acc[:, 0:64] / cnt
    npts_ref[...] = cnt


_final_call = pl.pallas_call(
    _final_body,
    out_shape=[
        jax.ShapeDtypeStruct((_K, _D), jnp.float32),
        jax.ShapeDtypeStruct((_K, 1), jnp.float32),
    ],
)


def kernel(x):
    ones = jnp.ones((_N, 1), jnp.float32)
    zeros = jnp.zeros((_N, _E - _D - 1), jnp.float32)
    x_ext = jnp.concatenate([x, ones, zeros], axis=1)
    acc = jnp.concatenate(
        [x[:_K], jnp.ones((_K, 1), jnp.float32),
         jnp.zeros((_K, _E - _D - 1), jnp.float32)], axis=1)
    clusters = None
    for _ in range(_NITERS):
        clusters, acc = _iter_call(x_ext, acc)
    cent, npts = _final_call(acc)
    return clusters, cent, npts.reshape(_K)


# speedup vs baseline: 1.1489x; 1.1489x over previous
import functools

import jax
import jax.numpy as jnp
from jax import lax
from jax.experimental import pallas as pl
from jax.experimental.pallas import tpu as pltpu
from jax.experimental.pallas import tpu_sc as plsc

_N = 65536
_D = 64
_K = 1024
_KP = 128
_BN = 2048
_NB = _N // _BN
_NW = 8
_WPTS = _N // _NW
_CH = 128
_NCH = _WPTS // _CH
_NCHT = _N // _CH
_HK = _K // 2
_NITERS = 10


def _merge_sums(acc_ref):
    top = acc_ref[0]
    bot = acc_ref[1]
    for w in range(1, _NW):
        top = top + acc_ref[2 * w]
        bot = bot + acc_ref[2 * w + 1]
    return jnp.concatenate([top, bot], axis=0)


def _iter_body(x_ref, acc_in_ref, cnt_in_ref, clusters_ref, adj_ref,
               cnt_out_ref, cent_ref, csq_ref):
    i = pl.program_id(0)

    @pl.when(i == 0)
    def _init():
        sums = _merge_sums(acc_in_ref)
        cent = sums[:, 0:_D] / cnt_in_ref[...]
        cent_ref[...] = cent
        csq_ref[...] = jnp.sum(cent * cent, axis=1)[None, :]

    xs = x_ref[...]
    cent = cent_ref[...]
    x_sq = jnp.sum(xs * xs, axis=1, keepdims=True)
    xc = jax.lax.dot_general(xs, cent, (((1,), (1,)), ((), ())),
                             preferred_element_type=jnp.float32)
    scores = (x_sq - 2.0 * xc) + csq_ref[...]
    idx = jnp.argmin(scores, axis=1).astype(jnp.int32)
    clusters_ref[...] = idx
    adj_ref[...] = idx + ((i // 4) % 4) * _K
    ohT = (jax.lax.broadcasted_iota(jnp.int32, (_K, _BN), 0)
           == idx[None, :]).astype(jnp.float32)
    pcnt = jax.lax.dot_general(ohT, jnp.ones((_BN, 1), jnp.float32),
                               (((1,), (0,)), ((), ())),
                               preferred_element_type=jnp.float32)

    @pl.when(i == 0)
    def _first():
        cnt_out_ref[...] = pcnt

    @pl.when(i > 0)
    def _rest():
        cnt_out_ref[...] += pcnt


_iter_call = pl.pallas_call(
    _iter_body,
    grid=(_NB,),
    in_specs=[
        pl.BlockSpec((_BN, _D), lambda i: (i, 0)),
        pl.BlockSpec((2 * _NW, _HK, _KP), lambda i: (0, 0, 0)),
        pl.BlockSpec((_K, 1), lambda i: (0, 0)),
    ],
    out_specs=[
        pl.BlockSpec((_BN,), lambda i: (i,)),
        pl.BlockSpec((_BN,), lambda i: (i,)),
        pl.BlockSpec((_K, 1), lambda i: (0, 0)),
    ],
    out_shape=[
        jax.ShapeDtypeStruct((_N,), jnp.int32),
        jax.ShapeDtypeStruct((_N,), jnp.int32),
        jax.ShapeDtypeStruct((_K, 1), jnp.float32),
    ],
    scratch_shapes=[
        pltpu.VMEM((_K, _D), jnp.float32),
        pltpu.VMEM((1, _K), jnp.float32),
    ],
    compiler_params=pltpu.CompilerParams(
        dimension_semantics=("arbitrary",),
    ),
)


_sc_mesh = plsc.VectorSubcoreMesh(core_axis_name="c", subcore_axis_name="s")


@functools.partial(
    pl.kernel,
    mesh=_sc_mesh,
    out_type=jax.ShapeDtypeStruct((2 * _NW, _HK, _KP), jnp.float32),
    scratch_types=[
        pltpu.VMEM((_CH,), jnp.int32),
        pltpu.VMEM((_CH, _KP), jnp.float32),
        pltpu.VMEM((_HK, _KP), jnp.float32),
        pltpu.VMEM_SHARED((4 * _K, _KP), jnp.float32),
    ],
)
def _sc_scatter(xp_hbm, adj_hbm, zeros_hbm, out_hbm, idx_v, rows_v, buf_v,
                acc_sh):
    c = lax.axis_index("c")
    s = lax.axis_index("s")

    @pl.when(s < 4)
    def _work():
        wid = c * 4 + s
        pltpu.sync_copy(zeros_hbm, buf_v)
        pltpu.sync_copy(buf_v, acc_sh.at[pl.ds(s * _K, _HK)])
        pltpu.sync_copy(buf_v, acc_sh.at[pl.ds(s * _K + _HK, _HK)])
        for ch in range(_NCH):
            j = wid * _NCH + ch
            pltpu.sync_copy(adj_hbm.at[j], idx_v)
            pltpu.sync_copy(xp_hbm.at[j], rows_v)
            pltpu.sync_copy(rows_v, acc_sh.at[idx_v], add=True)
        for h in range(2):
            pltpu.sync_copy(acc_sh.at[pl.ds(s * _K + h * _HK, _HK)], buf_v)
            pltpu.sync_copy(buf_v, out_hbm.at[2 * wid + h])


def _final_body(acc_ref, cnt_ref, cent_ref, npts_ref):
    sums = _merge_sums(acc_ref)
    cnt = cnt_ref[...]
    cent_ref[...] = sums[:, 0:_D] / cnt
    npts_ref[...] = cnt


_final_call = pl.pallas_call(
    _final_body,
    out_shape=[
        jax.ShapeDtypeStruct((_K, _D), jnp.float32),
        jax.ShapeDtypeStruct((_K, 1), jnp.float32),
    ],
)


def kernel(x):
    xp = jnp.concatenate(
        [x, jnp.zeros((_N, _KP - _D), jnp.float32)], axis=1)
    xp3 = xp.reshape(_NCHT, _CH, _KP)
    zrows = jnp.zeros((_HK, _KP), jnp.float32)
    seed = jnp.concatenate(
        [x[:_K], jnp.zeros((_K, _KP - _D), jnp.float32)],
        axis=1).reshape(2, _HK, _KP)
    acc = jnp.zeros((2 * _NW, _HK, _KP), jnp.float32).at[0:2].set(seed)
    cnt = jnp.ones((_K, 1), jnp.float32)
    clusters = None
    for _ in range(_NITERS):
        clusters, adj, cnt_new = _iter_call(x, acc, cnt)
        acc = _sc_scatter(xp3, adj.reshape(_NCHT, _CH), zrows)
        cnt = cnt_new
    cent, npts = _final_call(acc, cnt)
    return clusters, cent, npts.reshape(_K)


# speedup vs baseline: 1.4322x; 1.2466x over previous
import functools

import jax
import jax.numpy as jnp
from jax import lax
from jax.experimental import pallas as pl
from jax.experimental.pallas import tpu as pltpu
from jax.experimental.pallas import tpu_sc as plsc

_N = 65536
_D = 64
_K = 1024
_KP = 128
_BN = 2048
_NB = _N // _BN
_NW = 8
_WPTS = _N // _NW
_CH = 128
_NCH = _WPTS // _CH
_NCHT = _N // _CH
_HK = _K // 2
_NITERS = 10


def _merge_sums(acc_ref):
    parts = []
    for q in range(8):
        t = acc_ref[q]
        for w in range(1, _NW):
            t = t + acc_ref[8 * w + q]
        parts.append(t)
    return jnp.concatenate(parts, axis=0)


def _iter_body(x_ref, acc_in_ref, cnt_in_ref, clusters_ref, adj_ref,
               cnt_out_ref, cent_ref, csq_ref):
    i = pl.program_id(0)

    @pl.when(i == 0)
    def _init():
        sums = _merge_sums(acc_in_ref)
        cent = sums[:, 0:_D] / cnt_in_ref[...]
        cent_ref[...] = cent
        csq_ref[...] = jnp.sum(cent * cent, axis=1)[None, :]

    xs = x_ref[...]
    cent = cent_ref[...]
    x_sq = jnp.sum(xs * xs, axis=1, keepdims=True)
    xc = jax.lax.dot_general(xs, cent, (((1,), (1,)), ((), ())),
                             preferred_element_type=jnp.float32)
    scores = (x_sq - 2.0 * xc) + csq_ref[...]
    idx = jnp.argmin(scores, axis=1).astype(jnp.int32)
    clusters_ref[...] = idx
    adj_ref[...] = idx + ((i // 4) % 4) * _K
    ohT = (jax.lax.broadcasted_iota(jnp.int32, (_K, _BN), 0)
           == idx[None, :]).astype(jnp.float32)
    pcnt = jax.lax.dot_general(ohT, jnp.ones((_BN, 1), jnp.float32),
                               (((1,), (0,)), ((), ())),
                               preferred_element_type=jnp.float32)

    @pl.when(i == 0)
    def _first():
        cnt_out_ref[...] = pcnt

    @pl.when(i > 0)
    def _rest():
        cnt_out_ref[...] += pcnt


_iter_call = pl.pallas_call(
    _iter_body,
    grid=(_NB,),
    in_specs=[
        pl.BlockSpec((_BN, _D), lambda i: (i, 0)),
        pl.BlockSpec((8 * _NW, _CH, _KP), lambda i: (0, 0, 0)),
        pl.BlockSpec((_K, 1), lambda i: (0, 0)),
    ],
    out_specs=[
        pl.BlockSpec((_BN,), lambda i: (i,)),
        pl.BlockSpec((_BN,), lambda i: (i,)),
        pl.BlockSpec((_K, 1), lambda i: (0, 0)),
    ],
    out_shape=[
        jax.ShapeDtypeStruct((_N,), jnp.int32),
        jax.ShapeDtypeStruct((_N,), jnp.int32),
        jax.ShapeDtypeStruct((_K, 1), jnp.float32),
    ],
    scratch_shapes=[
        pltpu.VMEM((_K, _D), jnp.float32),
        pltpu.VMEM((1, _K), jnp.float32),
    ],
    compiler_params=pltpu.CompilerParams(
        dimension_semantics=("arbitrary",),
    ),
)


_sc_mesh = plsc.VectorSubcoreMesh(core_axis_name="c", subcore_axis_name="s")


@functools.partial(
    pl.kernel,
    mesh=_sc_mesh,
    out_type=jax.ShapeDtypeStruct((8 * _NW, _CH, _KP), jnp.float32),
    scratch_types=[
        pltpu.VMEM((_CH,), jnp.int32),
        pltpu.VMEM((_CH,), jnp.int32),
        pltpu.VMEM((_CH, _KP), jnp.float32),
        pltpu.VMEM((_CH, _KP), jnp.float32),
        pltpu.VMEM_SHARED((4 * _K, _KP), jnp.float32),
        pltpu.SemaphoreType.DMA,
    ],
)
def _sc_scatter(xp_hbm, adj_hbm, zeros_hbm, out_hbm, idx_v0, idx_v1,
                rows_v0, rows_v1, acc_sh, sem0):
    c = lax.axis_index("c")
    s = lax.axis_index("s")
    idx_b = (idx_v0, idx_v1)
    rows_b = (rows_v0, rows_v1)
    sem_b = (sem0, sem0)

    @pl.when(s < 4)
    def _work():
        wid = c * 4 + s
        pltpu.sync_copy(zeros_hbm, rows_v0)
        for q in range(_K // _CH):
            pltpu.sync_copy(rows_v0, acc_sh.at[pl.ds(s * _K + q * _CH, _CH)])
        j0 = wid * _NCH
        h_i = pltpu.async_copy(adj_hbm.at[j0], idx_b[0], sem_b[0])
        h_r = pltpu.async_copy(xp_hbm.at[j0], rows_b[0], sem_b[0])
        pending = (h_i, h_r)
        for ch in range(_NCH):
            cur = ch % 2
            nxt = 1 - cur
            pending[0].wait()
            pending[1].wait()
            if ch + 1 < _NCH:
                j = wid * _NCH + ch + 1
                h_i = pltpu.async_copy(adj_hbm.at[j], idx_b[nxt], sem_b[nxt])
                h_r = pltpu.async_copy(xp_hbm.at[j], rows_b[nxt], sem_b[nxt])
                pending = (h_i, h_r)
            pltpu.sync_copy(rows_b[cur], acc_sh.at[idx_b[cur]], add=True)
        for q in range(_K // _CH):
            pltpu.sync_copy(acc_sh.at[pl.ds(s * _K + q * _CH, _CH)], rows_v0)
            pltpu.sync_copy(rows_v0, out_hbm.at[(_K // _CH) * wid + q])


def _final_body(acc_ref, cnt_ref, cent_ref, npts_ref):
    sums = _merge_sums(acc_ref)
    cnt = cnt_ref[...]
    cent_ref[...] = sums[:, 0:_D] / cnt
    npts_ref[...] = cnt


_final_call = pl.pallas_call(
    _final_body,
    out_shape=[
        jax.ShapeDtypeStruct((_K, _D), jnp.float32),
        jax.ShapeDtypeStruct((_K, 1), jnp.float32),
    ],
)


def kernel(x):
    xp = jnp.concatenate(
        [x, jnp.zeros((_N, _KP - _D), jnp.float32)], axis=1)
    xp3 = xp.reshape(_NCHT, _CH, _KP)
    zrows = jnp.zeros((_CH, _KP), jnp.float32)
    seed = jnp.concatenate(
        [x[:_K], jnp.zeros((_K, _KP - _D), jnp.float32)],
        axis=1).reshape(8, _CH, _KP)
    acc = jnp.zeros((8 * _NW, _CH, _KP), jnp.float32).at[0:8].set(seed)
    cnt = jnp.ones((_K, 1), jnp.float32)
    clusters = None
    for _ in range(_NITERS):
        clusters, adj, cnt_new = _iter_call(x, acc, cnt)
        acc = _sc_scatter(xp3, adj.reshape(_NCHT, _CH), zrows)
        cnt = cnt_new
    cent, npts = _final_call(acc, cnt)
    return clusters, cent, npts.reshape(_K)
